# baseline (speedup 1.0000x reference)
import jax
import jax.numpy as jnp
from jax import lax
from jax.experimental import pallas as pl
from jax.experimental.pallas import tpu as pltpu
from jax.experimental.pallas import tpu_sc as plsc

VOCAB = 1_000_000
DIM = 32
BATCH = 4096
HIST = 200

NUM_CORES = 2
NUM_SUBCORES = 16
NW = NUM_CORES * NUM_SUBCORES
ROWS_PER_W = BATCH // NW
RCHUNK = 8
CHUNK = RCHUNK * HIST
NCHUNK = ROWS_PER_W // RCHUNK
NBUF = 2


def _emb_body(table_hbm, idx_hbm, out_hbm, idx_v, rows_v,
              isem0, isem1, gsem0, gsem1, osem0, osem1):
    isems = (isem0, isem1)
    gsems = (gsem0, gsem1)
    osems = (osem0, osem1)

    wid = lax.axis_index("s") * NUM_CORES + lax.axis_index("c")
    base_row = wid * ROWS_PER_W

    def idx_copies(b, g):
        r0 = base_row + g * RCHUNK
        return [
            pltpu.make_async_copy(
                idx_hbm.at[r0 + k], idx_v.at[b, pl.ds(k * HIST, HIST)],
                isems[b])
            for k in range(RCHUNK)
        ]

    def gather(b):
        return pltpu.make_async_copy(
            table_hbm.at[idx_v.at[b]], rows_v.at[b], gsems[b])

    def out_copies(b, g):
        r0 = base_row + g * RCHUNK
        return [
            pltpu.make_async_copy(
                rows_v.at[b, pl.ds(k * HIST, HIST), :], out_hbm.at[r0 + k],
                osems[b])
            for k in range(RCHUNK)
        ]

    for c in idx_copies(0, 0):
        c.start()
    for c in idx_copies(1, 1):
        c.start()
    for c in idx_copies(0, 0):
        c.wait()
    gather(0).start()

    def outer(gb, carry):
        for b in range(NBUF):
            g = gb + b
            bo = 1 - b

            @pl.when(g >= 1)
            def _():
                for c in out_copies(bo, g - 1):
                    c.wait()

            @pl.when(g + 1 < NCHUNK)
            def _():
                for c in idx_copies(bo, g + 1):
                    c.wait()
                gather(bo).start()

            gather(b).wait()
            for c in out_copies(b, g):
                c.start()

            @pl.when(g + 2 < NCHUNK)
            def _():
                for c in idx_copies(b, g + 2):
                    c.start()
        return carry

    lax.fori_loop(0, NCHUNK // NBUF, lambda i, c: outer(i * NBUF, c), 0,
                  unroll=False)

    for c in out_copies(1, NCHUNK - 1):
        c.wait()


_emb = pl.kernel(
    _emb_body,
    out_type=jax.ShapeDtypeStruct((BATCH, HIST, DIM), jnp.float32),
    mesh=plsc.VectorSubcoreMesh(core_axis_name="c", subcore_axis_name="s"),
    scratch_types=[
        pltpu.VMEM((NBUF, CHUNK), jnp.int32),
        pltpu.VMEM((NBUF, CHUNK, DIM), jnp.float32),
        pltpu.SemaphoreType.DMA,
        pltpu.SemaphoreType.DMA,
        pltpu.SemaphoreType.DMA,
        pltpu.SemaphoreType.DMA,
        pltpu.SemaphoreType.DMA,
        pltpu.SemaphoreType.DMA,
    ],
    compiler_params=pltpu.CompilerParams(use_tc_tiling_on_sc=False),
)


def kernel(inputs, embedding_table):
    return _emb(embedding_table, inputs.astype(jnp.int32))
